# Initial kernel scaffold; baseline (speedup 1.0000x reference)
import jax
import jax.numpy as jnp
from jax.experimental import pallas as pl


def kernel(projected_patches, pos_embed_table):
    raise NotImplementedError("write your pallas kernel here")


# speedup vs baseline: 1.0131x; 1.0131x over previous
import jax
import jax.numpy as jnp
from jax.experimental import pallas as pl

BATCH_BLOCK = 4


def _add_kernel(patches_ref, table_ref, out_ref):
    out_ref[...] = patches_ref[...] + table_ref[...]


def kernel(projected_patches, pos_embed_table):
    batch, num_patch, proj_dim = projected_patches.shape
    grid = (batch // BATCH_BLOCK,)
    return pl.pallas_call(
        _add_kernel,
        grid=grid,
        in_specs=[
            pl.BlockSpec((BATCH_BLOCK, num_patch, proj_dim), lambda i: (i, 0, 0)),
            pl.BlockSpec((num_patch, proj_dim), lambda i: (0, 0)),
        ],
        out_specs=pl.BlockSpec((BATCH_BLOCK, num_patch, proj_dim), lambda i: (i, 0, 0)),
        out_shape=jax.ShapeDtypeStruct(projected_patches.shape, projected_patches.dtype),
    )(projected_patches, pos_embed_table)


# speedup vs baseline: 1.0232x; 1.0099x over previous
import jax
import jax.numpy as jnp
from jax.experimental import pallas as pl

BATCH_BLOCK = 8


def _add_kernel(patches_ref, table_ref, out_ref):
    out_ref[...] = patches_ref[...] + table_ref[...]


def kernel(projected_patches, pos_embed_table):
    batch, num_patch, proj_dim = projected_patches.shape
    grid = (batch // BATCH_BLOCK,)
    return pl.pallas_call(
        _add_kernel,
        grid=grid,
        in_specs=[
            pl.BlockSpec((BATCH_BLOCK, num_patch, proj_dim), lambda i: (i, 0, 0)),
            pl.BlockSpec((num_patch, proj_dim), lambda i: (0, 0)),
        ],
        out_specs=pl.BlockSpec((BATCH_BLOCK, num_patch, proj_dim), lambda i: (i, 0, 0)),
        out_shape=jax.ShapeDtypeStruct(projected_patches.shape, projected_patches.dtype),
    )(projected_patches, pos_embed_table)
